# baseline (speedup 1.0000x reference)
import jax
import jax.numpy as jnp
from jax import lax
from jax.experimental import pallas as pl
from jax.experimental.pallas import tpu as pltpu
from jax.experimental.pallas import tpu_sc as plsc

N_NODES = 10000
N_EDGES = 320000
NUM_REL = 64
DIM = 128

NUM_CORES = 2
NUM_SUBCORES = 16

PAD_EDGES = 327680
IDX_ROWS = PAD_EDGES // DIM
MACRO_ROWS = 8
N_MACROS = IDX_ROWS // MACRO_ROWS
MACROS_PER_TILE = N_MACROS // NUM_SUBCORES

ACC_ROWS = 10016
DUMP_NODE = N_NODES
SROWS_TILE = 640


DEG_ROWS = 640


def _sc_body(gidx_hbm, sidx_hbm, typ_hbm, ent_hbm, negrel_hbm, eye_hbm,
             s_out, d_out, acc_s, acc_d, negrel_s, ohd_s,
             rows_v, rrows_v, gi_v, si_v, ty_v, ki_v, di_v, sem, semb):
    c = lax.axis_index("c")
    w = lax.axis_index("s")
    zeros16 = jnp.zeros((16,), jnp.float32)

    def _zrows(i, carry):
        for j in range(8):
            rows_v[i, pl.ds(16 * j, 16)] = zeros16
        return carry
    lax.fori_loop(0, DIM, _zrows, 0)

    base = w * SROWS_TILE
    pltpu.sync_copy(rows_v.at[pl.ds(0, DEG_ROWS // NUM_SUBCORES)],
                    acc_d.at[pl.ds(w * (DEG_ROWS // NUM_SUBCORES),
                                   DEG_ROWS // NUM_SUBCORES)])

    @pl.when(w < NUM_SUBCORES - 1)
    def _():
        for k in range(5):
            pltpu.sync_copy(rows_v, acc_s.at[pl.ds(base + DIM * k, DIM)])

    @pl.when(w == NUM_SUBCORES - 1)
    def _():
        for k in range(3):
            pltpu.sync_copy(rows_v, acc_s.at[pl.ds(base + DIM * k, DIM)])
        pltpu.sync_copy(rows_v.at[pl.ds(0, 32)],
                        acc_s.at[pl.ds(base + 3 * DIM, 32)])

    @pl.when(w == 0)
    def _():
        pltpu.sync_copy(negrel_hbm, rrows_v.at[pl.ds(0, NUM_REL)])
        pltpu.sync_copy(rrows_v.at[pl.ds(0, NUM_REL)], negrel_s)
        pltpu.sync_copy(eye_hbm, rrows_v.at[pl.ds(0, 16)])
        pltpu.sync_copy(rrows_v.at[pl.ds(0, 16)], ohd_s)
    plsc.subcore_barrier()

    def _step(i, carry):
        m = w + NUM_SUBCORES * i
        r0 = MACRO_ROWS * m
        pltpu.sync_copy(gidx_hbm.at[c, pl.ds(r0, MACRO_ROWS)], gi_v)
        pltpu.sync_copy(sidx_hbm.at[c, pl.ds(r0, MACRO_ROWS)], si_v)
        pltpu.sync_copy(typ_hbm.at[pl.ds(r0, MACRO_ROWS)], ty_v)
        for j in range(MACRO_ROWS):
            for t in range(8):
                s = si_v[j, pl.ds(16 * t, 16)]
                ki_v[j, pl.ds(16 * t, 16)] = s & 15
                di_v[j, pl.ds(16 * t, 16)] = s >> 4
        for j in range(MACRO_ROWS):
            cpe = pltpu.async_copy(ent_hbm.at[gi_v.at[j]], rows_v, sem)
            pltpu.async_copy(negrel_s.at[ty_v.at[j]], rrows_v, semb).wait()
            pltpu.sync_copy(rrows_v, acc_s.at[si_v.at[j]], add=True)
            pltpu.async_copy(ohd_s.at[ki_v.at[j]], rrows_v, semb).wait()
            pltpu.sync_copy(rrows_v, acc_d.at[di_v.at[j]], add=True)
            cpe.wait()
            pltpu.sync_copy(rows_v, acc_s.at[si_v.at[j]], add=True)
        return carry
    lax.fori_loop(0, MACROS_PER_TILE, _step, 0)

    plsc.subcore_barrier()
    pltpu.sync_copy(acc_d.at[pl.ds(w * (DEG_ROWS // NUM_SUBCORES),
                                   DEG_ROWS // NUM_SUBCORES)],
                    d_out.at[c, pl.ds(w * (DEG_ROWS // NUM_SUBCORES),
                                      DEG_ROWS // NUM_SUBCORES)])

    @pl.when(w < NUM_SUBCORES - 1)
    def _():
        pltpu.sync_copy(acc_s.at[pl.ds(base, SROWS_TILE)],
                        s_out.at[c, pl.ds(base, SROWS_TILE)])

    @pl.when(w == NUM_SUBCORES - 1)
    def _():
        pltpu.sync_copy(acc_s.at[pl.ds(base, N_NODES - 15 * SROWS_TILE)],
                        s_out.at[c, pl.ds(base, N_NODES - 15 * SROWS_TILE)])


def _sc_aggregate(gidx, sidx, typ2d, entity, negrel, eye16):
    mesh = plsc.VectorSubcoreMesh(core_axis_name="c", subcore_axis_name="s",
                                  num_cores=NUM_CORES,
                                  num_subcores=NUM_SUBCORES)
    return pl.kernel(
        _sc_body,
        out_type=(
            jax.ShapeDtypeStruct((NUM_CORES, N_NODES, DIM), jnp.float32),
            jax.ShapeDtypeStruct((NUM_CORES, DEG_ROWS, DIM), jnp.float32),
        ),
        mesh=mesh,
        scratch_types=[
            pltpu.VMEM_SHARED((ACC_ROWS, DIM), jnp.float32),
            pltpu.VMEM_SHARED((DEG_ROWS, DIM), jnp.float32),
            pltpu.VMEM_SHARED((NUM_REL, DIM), jnp.float32),
            pltpu.VMEM_SHARED((16, DIM), jnp.float32),
            pltpu.VMEM((DIM, DIM), jnp.float32),
            pltpu.VMEM((DIM, DIM), jnp.float32),
            pltpu.VMEM((MACRO_ROWS, DIM), jnp.int32),
            pltpu.VMEM((MACRO_ROWS, DIM), jnp.int32),
            pltpu.VMEM((MACRO_ROWS, DIM), jnp.int32),
            pltpu.VMEM((MACRO_ROWS, DIM), jnp.int32),
            pltpu.VMEM((MACRO_ROWS, DIM), jnp.int32),
            pltpu.SemaphoreType.DMA,
            pltpu.SemaphoreType.DMA,
        ],
    )(gidx, sidx, typ2d, entity, negrel, eye16)


def _dg(a, b):
    return lax.dot_general(a, b, (((1,), (1,)), ((), ())),
                           preferred_element_type=jnp.float32)


def _tc_body(ent, s_acc, deg_in, rel, w_o, w_i, w_s, w_rel, gamma, beta,
             out_ref, rel_out_ref):
    acc = _dg(ent[...], w_s[...])
    acc += _dg(s_acc[0], w_o[...])
    acc += _dg(s_acc[1], w_i[...])
    deg = jnp.maximum(deg_in[...], 1.0)
    acc = acc / deg
    mean = jnp.mean(acc, axis=0, keepdims=True)
    ctr = acc - mean
    var = jnp.mean(ctr * ctr, axis=0, keepdims=True)
    acc = ctr * lax.rsqrt(var + 1e-5) * gamma[...] + beta[...]
    out_ref[...] = jnp.maximum(acc, 0.0)
    rel_out_ref[...] = _dg(rel[...], w_rel[...])


def _tc_finish(entity, s_acc, deg, rel, w_o, w_i, w_s, w_rel, gamma, beta):
    return pl.pallas_call(
        _tc_body,
        out_shape=(
            jax.ShapeDtypeStruct((N_NODES, DIM), jnp.float32),
            jax.ShapeDtypeStruct((NUM_REL, DIM), jnp.float32),
        ),
    )(entity, s_acc, deg, rel, w_o, w_i, w_s, w_rel, gamma, beta)


def kernel(entity_emb, relation_emb, edge_index, edge_type, W_O, W_I, W_S,
           W_rel, gamma, beta):
    src = edge_index[0]
    dst = edge_index[1]
    npad = PAD_EDGES - N_EDGES
    zpad = jnp.zeros((npad,), jnp.int32)
    dpad = jnp.full((npad,), DUMP_NODE, jnp.int32)
    gidx = jnp.stack([jnp.concatenate([src, zpad]),
                      jnp.concatenate([dst, zpad])]).reshape(
                          NUM_CORES, IDX_ROWS, DIM)
    sidx = jnp.stack([jnp.concatenate([dst, dpad]),
                      jnp.concatenate([src, dpad])]).reshape(
                          NUM_CORES, IDX_ROWS, DIM)
    typ2d = jnp.concatenate([edge_type, zpad]).reshape(IDX_ROWS, DIM)
    negrel = -relation_emb
    eye16 = jnp.eye(16, DIM, dtype=jnp.float32)

    s_acc, d_raw = _sc_aggregate(gidx, sidx, typ2d, entity_emb, negrel,
                                 eye16)
    deg = (d_raw[0, :, :16] + d_raw[1, :, :16]).reshape(
        NUM_SUBCORES * DEG_ROWS)[:N_NODES].reshape(N_NODES, 1)

    out, new_rel = _tc_finish(entity_emb, s_acc, deg, relation_emb,
                              W_O, W_I, W_S, W_rel,
                              gamma.reshape(1, DIM), beta.reshape(1, DIM))
    return (out, new_rel)


# speedup vs baseline: 1.4060x; 1.4060x over previous
import jax
import jax.numpy as jnp
from jax import lax
from jax.experimental import pallas as pl
from jax.experimental.pallas import tpu as pltpu
from jax.experimental.pallas import tpu_sc as plsc

N_NODES = 10000
N_EDGES = 320000
NUM_REL = 64
DIM = 128

NUM_CORES = 2
NUM_SUBCORES = 16

PAD_EDGES = 327680
IDX_ROWS = PAD_EDGES // DIM
MACRO_ROWS = 8
N_MACROS = IDX_ROWS // MACRO_ROWS
MACROS_PER_TILE = N_MACROS // NUM_SUBCORES

ACC_ROWS = 10016
DUMP_NODE = N_NODES
SROWS_TILE = 640

CACC_ROWS = 5120
CROWS_TILE = CACC_ROWS // NUM_SUBCORES


def _zero_vmem(ref, zeros16):
    def _z(i, carry):
        for j in range(8):
            ref[i, pl.ds(16 * j, 16)] = zeros16
        return carry
    lax.fori_loop(0, DIM, _z, 0)


def _ent_body(gidx_hbm, sidx_hbm, ent_hbm, s_out,
              acc_s, rows_v, rows_w, gi_v, si_v, sem, semb):
    c = lax.axis_index("c")
    w = lax.axis_index("s")
    _zero_vmem(rows_v, jnp.zeros((16,), jnp.float32))

    base = w * SROWS_TILE

    @pl.when(w < NUM_SUBCORES - 1)
    def _():
        for k in range(5):
            pltpu.sync_copy(rows_v, acc_s.at[pl.ds(base + DIM * k, DIM)])

    @pl.when(w == NUM_SUBCORES - 1)
    def _():
        for k in range(3):
            pltpu.sync_copy(rows_v, acc_s.at[pl.ds(base + DIM * k, DIM)])
        pltpu.sync_copy(rows_v.at[pl.ds(0, 32)],
                        acc_s.at[pl.ds(base + 3 * DIM, 32)])
    plsc.subcore_barrier()

    def _step(i, carry):
        m = w + NUM_SUBCORES * i
        r0 = MACRO_ROWS * m
        pltpu.sync_copy(gidx_hbm.at[c, pl.ds(r0, MACRO_ROWS)], gi_v)
        pltpu.sync_copy(sidx_hbm.at[c, pl.ds(r0, MACRO_ROWS)], si_v)
        for h in range(MACRO_ROWS // 2):
            cpa = pltpu.async_copy(ent_hbm.at[gi_v.at[2 * h]], rows_v, sem)
            cpb = pltpu.async_copy(ent_hbm.at[gi_v.at[2 * h + 1]], rows_w,
                                   semb)
            cpa.wait()
            pltpu.sync_copy(rows_v, acc_s.at[si_v.at[2 * h]], add=True)
            cpb.wait()
            pltpu.sync_copy(rows_w, acc_s.at[si_v.at[2 * h + 1]], add=True)
        return carry
    lax.fori_loop(0, MACROS_PER_TILE, _step, 0)

    plsc.subcore_barrier()

    @pl.when(w < NUM_SUBCORES - 1)
    def _():
        pltpu.sync_copy(acc_s.at[pl.ds(base, SROWS_TILE)],
                        s_out.at[c, pl.ds(base, SROWS_TILE)])

    @pl.when(w == NUM_SUBCORES - 1)
    def _():
        pltpu.sync_copy(acc_s.at[pl.ds(base, N_NODES - 15 * SROWS_TILE)],
                        s_out.at[c, pl.ds(base, N_NODES - 15 * SROWS_TILE)])


def _cnt_body(sidx_hbm, typ_hbm, ohq_hbm, c_out,
              acc_c, ohq_s, cbuf, cbuf2, si_v, ty_v, fi_v, qi_v, sem, semb):
    c = lax.axis_index("c")
    w = lax.axis_index("s")
    _zero_vmem(cbuf, jnp.zeros((16,), jnp.float32))

    cbase = w * CROWS_TILE
    for k in range(2):
        pltpu.sync_copy(cbuf, acc_c.at[pl.ds(cbase + DIM * k, DIM)])
    pltpu.sync_copy(cbuf.at[pl.ds(0, CROWS_TILE - 2 * DIM)],
                    acc_c.at[pl.ds(cbase + 2 * DIM, CROWS_TILE - 2 * DIM)])

    @pl.when(w == 0)
    def _():
        pltpu.sync_copy(ohq_hbm, cbuf)
        pltpu.sync_copy(cbuf, ohq_s)
    plsc.subcore_barrier()

    def _step(i, carry):
        m = w + NUM_SUBCORES * i
        r0 = MACRO_ROWS * m
        pltpu.sync_copy(sidx_hbm.at[c, pl.ds(r0, MACRO_ROWS)], si_v)
        pltpu.sync_copy(typ_hbm.at[pl.ds(r0, MACRO_ROWS)], ty_v)
        for j in range(MACRO_ROWS):
            for t in range(8):
                s = si_v[j, pl.ds(16 * t, 16)]
                ty = ty_v[j, pl.ds(16 * t, 16)]
                fi_v[j, pl.ds(16 * t, 16)] = s >> 1
                qi_v[j, pl.ds(16 * t, 16)] = (s & 1) * NUM_REL + ty
        for h in range(MACRO_ROWS // 2):
            cpa = pltpu.async_copy(ohq_s.at[qi_v.at[2 * h]], cbuf, sem)
            cpb = pltpu.async_copy(ohq_s.at[qi_v.at[2 * h + 1]], cbuf2,
                                   semb)
            cpa.wait()
            pltpu.sync_copy(cbuf, acc_c.at[fi_v.at[2 * h]], add=True)
            cpb.wait()
            pltpu.sync_copy(cbuf2, acc_c.at[fi_v.at[2 * h + 1]], add=True)
        return carry
    lax.fori_loop(0, MACROS_PER_TILE, _step, 0)

    plsc.subcore_barrier()
    pltpu.sync_copy(acc_c.at[pl.ds(cbase, CROWS_TILE)],
                    c_out.at[c, pl.ds(cbase, CROWS_TILE)])


def _sc_entity(gidx, sidx, entity):
    mesh = plsc.VectorSubcoreMesh(core_axis_name="c", subcore_axis_name="s",
                                  num_cores=NUM_CORES,
                                  num_subcores=NUM_SUBCORES)
    return pl.kernel(
        _ent_body,
        out_type=jax.ShapeDtypeStruct((NUM_CORES, N_NODES, DIM),
                                      jnp.float32),
        mesh=mesh,
        scratch_types=[
            pltpu.VMEM_SHARED((ACC_ROWS, DIM), jnp.float32),
            pltpu.VMEM((DIM, DIM), jnp.float32),
            pltpu.VMEM((DIM, DIM), jnp.float32),
            pltpu.VMEM((MACRO_ROWS, DIM), jnp.int32),
            pltpu.VMEM((MACRO_ROWS, DIM), jnp.int32),
            pltpu.SemaphoreType.DMA,
            pltpu.SemaphoreType.DMA,
        ],
    )(gidx, sidx, entity)


def _sc_counts(sidx, typ2d, ohq):
    mesh = plsc.VectorSubcoreMesh(core_axis_name="c", subcore_axis_name="s",
                                  num_cores=NUM_CORES,
                                  num_subcores=NUM_SUBCORES)
    return pl.kernel(
        _cnt_body,
        out_type=jax.ShapeDtypeStruct((NUM_CORES, CACC_ROWS, DIM),
                                      jnp.float32),
        mesh=mesh,
        scratch_types=[
            pltpu.VMEM_SHARED((CACC_ROWS, DIM), jnp.float32),
            pltpu.VMEM_SHARED((DIM, DIM), jnp.float32),
            pltpu.VMEM((DIM, DIM), jnp.float32),
            pltpu.VMEM((DIM, DIM), jnp.float32),
            pltpu.VMEM((MACRO_ROWS, DIM), jnp.int32),
            pltpu.VMEM((MACRO_ROWS, DIM), jnp.int32),
            pltpu.VMEM((MACRO_ROWS, DIM), jnp.int32),
            pltpu.VMEM((MACRO_ROWS, DIM), jnp.int32),
            pltpu.SemaphoreType.DMA,
            pltpu.SemaphoreType.DMA,
        ],
    )(sidx, typ2d, ohq)


def _dg(a, b):
    return lax.dot_general(a, b, (((1,), (1,)), ((), ())),
                           preferred_element_type=jnp.float32)


def _tc_body(ent, s_acc, c_acc, rel, w_o, w_i, w_s, w_rel, gamma, beta,
             out_ref, rel_out_ref):
    c1 = c_acc[0]
    c2 = c_acc[1]
    p_o = _dg(rel[...], w_o[...])
    p_i = _dg(rel[...], w_i[...])
    acc = _dg(ent[...], w_s[...])
    acc += _dg(s_acc[0], w_o[...])
    acc += _dg(s_acc[1], w_i[...])
    acc -= lax.dot_general(c1, p_o, (((1,), (0,)), ((), ())),
                           preferred_element_type=jnp.float32)
    acc -= lax.dot_general(c2, p_i, (((1,), (0,)), ((), ())),
                           preferred_element_type=jnp.float32)
    deg = jnp.sum(c1, axis=1, keepdims=True) + jnp.sum(c2, axis=1,
                                                       keepdims=True)
    deg = jnp.maximum(deg, 1.0)
    acc = acc / deg
    mean = jnp.mean(acc, axis=0, keepdims=True)
    ctr = acc - mean
    var = jnp.mean(ctr * ctr, axis=0, keepdims=True)
    acc = ctr * lax.rsqrt(var + 1e-5) * gamma[...] + beta[...]
    out_ref[...] = jnp.maximum(acc, 0.0)
    rel_out_ref[...] = _dg(rel[...], w_rel[...])


def _tc_finish(entity, s_acc, c_acc, rel, w_o, w_i, w_s, w_rel, gamma, beta):
    return pl.pallas_call(
        _tc_body,
        out_shape=(
            jax.ShapeDtypeStruct((N_NODES, DIM), jnp.float32),
            jax.ShapeDtypeStruct((NUM_REL, DIM), jnp.float32),
        ),
    )(entity, s_acc, c_acc, rel, w_o, w_i, w_s, w_rel, gamma, beta)


def kernel(entity_emb, relation_emb, edge_index, edge_type, W_O, W_I, W_S,
           W_rel, gamma, beta):
    src = edge_index[0]
    dst = edge_index[1]
    npad = PAD_EDGES - N_EDGES
    zpad = jnp.zeros((npad,), jnp.int32)
    dpad = jnp.full((npad,), DUMP_NODE, jnp.int32)
    gidx = jnp.stack([jnp.concatenate([src, zpad]),
                      jnp.concatenate([dst, zpad])]).reshape(
                          NUM_CORES, IDX_ROWS, DIM)
    sidx = jnp.stack([jnp.concatenate([dst, dpad]),
                      jnp.concatenate([src, dpad])]).reshape(
                          NUM_CORES, IDX_ROWS, DIM)
    typ2d = jnp.concatenate([edge_type, zpad]).reshape(IDX_ROWS, DIM)
    ohq = jnp.eye(DIM, dtype=jnp.float32)

    s_acc = _sc_entity(gidx, sidx, entity_emb)
    c_pad = _sc_counts(sidx, typ2d, ohq)
    c_acc = c_pad.reshape(NUM_CORES, 2 * CACC_ROWS, NUM_REL)[:, :N_NODES, :]

    out, new_rel = _tc_finish(entity_emb, s_acc, c_acc, relation_emb,
                              W_O, W_I, W_S, W_rel,
                              gamma.reshape(1, DIM), beta.reshape(1, DIM))
    return (out, new_rel)
